# baseline (speedup 1.0000x reference)
import jax
import jax.numpy as jnp
from jax import lax
from jax.experimental import pallas as pl
from jax.experimental.pallas import tpu as pltpu

N_TOKENS = 32768
D_MODEL = 768
N_EXP = 64
BM = 2048


def _router_body(x_ref, w_ref, b_ref, ow_ref, oi_ref):
    x_blk = x_ref[...]
    w = w_ref[...]
    logits = lax.dot_general(
        x_blk, w, (((1,), (1,)), ((), ())),
        preferred_element_type=jnp.float32,
    ) + b_ref[...]

    idx = lax.broadcasted_iota(jnp.int32, (BM, N_EXP), 1)
    m1 = jnp.max(logits, axis=1, keepdims=True)
    i1 = jnp.min(jnp.where(logits == m1, idx, N_EXP), axis=1, keepdims=True)
    l2 = jnp.where(idx == i1, -jnp.inf, logits)
    m2 = jnp.max(l2, axis=1, keepdims=True)
    i2 = jnp.min(jnp.where(l2 == m2, idx, N_EXP), axis=1, keepdims=True)

    e = jnp.exp(m2 - m1)
    denom = 1.0 + e
    ow_ref[...] = jnp.concatenate([1.0 / denom, e / denom], axis=1)
    oi_ref[...] = jnp.concatenate([i1, i2], axis=1)


@jax.jit
def kernel(x, W, b):
    b2 = b.reshape(1, N_EXP)
    grid = (N_TOKENS // BM,)
    out_w, out_i = pl.pallas_call(
        _router_body,
        grid=grid,
        in_specs=[
            pl.BlockSpec((BM, D_MODEL), lambda i: (i, 0)),
            pl.BlockSpec((N_EXP, D_MODEL), lambda i: (0, 0)),
            pl.BlockSpec((1, N_EXP), lambda i: (0, 0)),
        ],
        out_specs=[
            pl.BlockSpec((BM, 2), lambda i: (i, 0)),
            pl.BlockSpec((BM, 2), lambda i: (i, 0)),
        ],
        out_shape=[
            jax.ShapeDtypeStruct((N_TOKENS, 2), jnp.float32),
            jax.ShapeDtypeStruct((N_TOKENS, 2), jnp.int32),
        ],
        compiler_params=pltpu.CompilerParams(
            dimension_semantics=("arbitrary",),
        ),
    )(x, W, b2)
    return out_w, out_i


# speedup vs baseline: 1.0770x; 1.0770x over previous
import jax
import jax.numpy as jnp
from jax import lax
from jax.experimental import pallas as pl
from jax.experimental.pallas import tpu as pltpu

N_TOKENS = 32768
D_MODEL = 768
N_EXP = 64
BM = 4096


def _router_body(x_ref, w_ref, b_ref, ow_ref, oi_ref):
    x_blk = x_ref[...]
    w = w_ref[...]
    logits = lax.dot_general(
        x_blk, w, (((1,), (1,)), ((), ())),
        preferred_element_type=jnp.float32,
    ) + b_ref[...]

    idx = lax.broadcasted_iota(jnp.int32, (BM, N_EXP), 1)
    m1 = jnp.max(logits, axis=1, keepdims=True)
    i1 = jnp.min(jnp.where(logits == m1, idx, N_EXP), axis=1, keepdims=True)
    l2 = jnp.where(idx == i1, -jnp.inf, logits)
    m2 = jnp.max(l2, axis=1, keepdims=True)
    i2 = jnp.min(jnp.where(l2 == m2, idx, N_EXP), axis=1, keepdims=True)

    e = jnp.exp(m2 - m1)
    denom = 1.0 + e
    ow_ref[...] = jnp.concatenate([1.0 / denom, e / denom], axis=1)
    oi_ref[...] = jnp.concatenate([i1, i2], axis=1)


@jax.jit
def kernel(x, W, b):
    b2 = b.reshape(1, N_EXP)
    grid = (N_TOKENS // BM,)
    out_w, out_i = pl.pallas_call(
        _router_body,
        grid=grid,
        in_specs=[
            pl.BlockSpec((BM, D_MODEL), lambda i: (i, 0)),
            pl.BlockSpec((N_EXP, D_MODEL), lambda i: (0, 0)),
            pl.BlockSpec((1, N_EXP), lambda i: (0, 0)),
        ],
        out_specs=[
            pl.BlockSpec((BM, 2), lambda i: (i, 0)),
            pl.BlockSpec((BM, 2), lambda i: (i, 0)),
        ],
        out_shape=[
            jax.ShapeDtypeStruct((N_TOKENS, 2), jnp.float32),
            jax.ShapeDtypeStruct((N_TOKENS, 2), jnp.int32),
        ],
        compiler_params=pltpu.CompilerParams(
            dimension_semantics=("parallel",),
        ),
    )(x, W, b2)
    return out_w, out_i
